# baseline (speedup 1.0000x reference)
import functools

import jax
import jax.numpy as jnp
from jax import lax
from jax.experimental import pallas as pl
from jax.experimental.pallas import tpu as pltpu
from jax.experimental.pallas import tpu_sc as plsc

N = 640000
D = 16
HIST = 100
ALPHA = 0.4
BETA = 0.4
GAMMA = 0.2
DECAY = 0.99

B = 25600
NB = N // B

NW = 32
CHUNK = 5120
NCH = N // CHUNK
NFULL = 12
FULL_JOBS = NFULL * NCH
NBUF = 3


def _sc_hist(hist_hbm, w_hbm, out_hbm, buf0, buf1, buf2,
             si0, si1, si2, so0, so1, so2):
    wid = lax.axis_index("s") * 2 + lax.axis_index("c")
    bufs = (buf0, buf1, buf2)
    sin = (si0, si1, si2)
    sout = (so0, so1, so2)

    def _in(j, p):
        stripe = j // NCH
        col = (j % NCH) * CHUNK
        o = pl.multiple_of(stripe * 8, 8)
        return pltpu.make_async_copy(
            hist_hbm.at[pl.ds(o, 8), pl.ds(col, CHUNK)], bufs[p], sin[p])

    def _out(j, p):
        stripe = j // NCH
        col = (j % NCH) * CHUNK
        o = pl.multiple_of(stripe * 8, 8)
        return pltpu.make_async_copy(
            bufs[p], out_hbm.at[pl.ds(o, 8), pl.ds(col, CHUNK)], sout[p])

    def group(g, carry):
        base = wid + NW * NBUF * g
        for p in range(NBUF):
            j = base + NW * p

            @pl.when(j < FULL_JOBS)
            def _(j=j, p=p):
                _in(j, p).start()
        for p in range(NBUF):
            j = base + NW * p

            @pl.when(j < FULL_JOBS)
            def _(j=j, p=p):
                _in(j, p).wait()

                @pl.when(j // NCH == 0)
                def _():
                    col = (j % NCH) * CHUNK
                    pltpu.async_copy(w_hbm.at[pl.ds(col, CHUNK)],
                                     bufs[p].at[0], sin[p]).wait()

                _out(j, p).start()
        for p in range(NBUF):
            j = base + NW * p

            @pl.when(j < FULL_JOBS)
            def _(j=j, p=p):
                _out(j, p).wait()
        return carry

    ngroups = (FULL_JOBS + NW * NBUF - 1) // (NW * NBUF)
    lax.fori_loop(0, ngroups, group, 0)

    def tail_body(k, carry):
        j = wid + NW * k

        @pl.when(j < NCH)
        def _():
            col = j * CHUNK
            pltpu.async_copy(
                hist_hbm.at[pl.ds(96, 4), pl.ds(col, CHUNK)],
                buf0.at[pl.ds(0, 4)], si0).wait()
            pltpu.async_copy(
                buf0.at[pl.ds(0, 4)],
                out_hbm.at[pl.ds(96, 4), pl.ds(col, CHUNK)], si0).wait()
        return carry

    lax.fori_loop(0, (NCH + NW - 1) // NW, tail_body, 0)


def _tc_body(g_ref, s_ref, t_ref, ge_ref, fe_ref,
             nge_ref, nfe_ref, u_ref, nge_s, nfe_s, smax):
    i = pl.program_id(0)

    @pl.when(i < NB)
    def _phase1():
        nge = DECAY * ge_ref[...] + (1.0 - DECAY) * jnp.abs(g_ref[...])
        nge_ref[...] = nge
        nge_s[pl.ds(i * B, B)] = nge
        bmax = jnp.max(nge)
        prev = jnp.where(i == 0, 0.0, smax[0])
        smax[0] = jnp.maximum(prev, bmax)

        s = s_ref[...]
        t = t_ref[...]
        sum_s = jnp.sum(s, axis=0)
        sum_t = jnp.sum(t, axis=0)
        sum_st = jnp.sum(s * t, axis=0)
        sum_ss = jnp.sum(s * s, axis=0)
        sum_tt = jnp.sum(t * t, axis=0)
        cov = sum_st - sum_s * sum_t * (1.0 / D)
        var_s = sum_ss - sum_s * sum_s * (1.0 / D)
        var_t = sum_tt - sum_t * sum_t * (1.0 / D)
        corr = cov / ((jnp.sqrt(var_s) + 1e-6) * (jnp.sqrt(var_t) + 1e-6))
        nfe = DECAY * fe_ref[...] + (1.0 - DECAY) * jnp.abs(corr)
        nfe_ref[...] = nfe
        nfe_s[pl.ds(i * B, B)] = nfe

    @pl.when(i >= NB)
    def _phase2():
        j = i - NB
        m = smax[0]
        u_ref[...] = (ALPHA / (m + 1e-6)) * nge_s[pl.ds(j * B, B)] \
            + (BETA * nfe_s[pl.ds(j * B, B)] + GAMMA)


def kernel(gradients, source_activations, target_activations, weights,
           gradient_ema, flow_ema, weight_history):
    sc_copy = pl.kernel(
        _sc_hist,
        out_type=jax.ShapeDtypeStruct((HIST, N), jnp.float32),
        mesh=plsc.VectorSubcoreMesh(core_axis_name="c", subcore_axis_name="s"),
        scratch_types=[pltpu.VMEM((8, CHUNK), jnp.float32),
                       pltpu.VMEM((8, CHUNK), jnp.float32),
                       pltpu.VMEM((8, CHUNK), jnp.float32),
                       pltpu.SemaphoreType.DMA, pltpu.SemaphoreType.DMA,
                       pltpu.SemaphoreType.DMA, pltpu.SemaphoreType.DMA,
                       pltpu.SemaphoreType.DMA, pltpu.SemaphoreType.DMA],
    )
    nhist = sc_copy(weight_history, weights)

    sT = source_activations.T
    tT = target_activations.T

    clamp = lambda i: (jnp.minimum(i, NB - 1),)
    clamp2 = lambda i: (0, jnp.minimum(i, NB - 1))
    nge, nfe, utility = pl.pallas_call(
        _tc_body,
        grid=(2 * NB,),
        in_specs=[
            pl.BlockSpec((B,), clamp),
            pl.BlockSpec((D, B), clamp2),
            pl.BlockSpec((D, B), clamp2),
            pl.BlockSpec((B,), clamp),
            pl.BlockSpec((B,), clamp),
        ],
        out_specs=[
            pl.BlockSpec((B,), clamp),
            pl.BlockSpec((B,), clamp),
            pl.BlockSpec((B,), lambda i: (jnp.maximum(i - NB, 0),)),
        ],
        out_shape=[
            jax.ShapeDtypeStruct((N,), jnp.float32),
            jax.ShapeDtypeStruct((N,), jnp.float32),
            jax.ShapeDtypeStruct((N,), jnp.float32),
        ],
        scratch_shapes=[
            pltpu.VMEM((N,), jnp.float32),
            pltpu.VMEM((N,), jnp.float32),
            pltpu.SMEM((1,), jnp.float32),
        ],
    )(gradients, sT, tT, gradient_ema, flow_ema)

    return (utility, nge, nfe, nhist)


# speedup vs baseline: 1.0620x; 1.0620x over previous
import functools

import jax
import jax.numpy as jnp
from jax import lax
from jax.experimental import pallas as pl
from jax.experimental.pallas import tpu as pltpu
from jax.experimental.pallas import tpu_sc as plsc

N = 640000
D = 16
HIST = 100
ALPHA = 0.4
BETA = 0.4
GAMMA = 0.2
DECAY = 0.99

B = 25600
NB = N // B

NW = 32
CHUNK = 6400
NCH = N // CHUNK
NFULL = 12
FULL_JOBS = NFULL * NCH
NBUF = 2


def _sc_hist(hist_hbm, w_hbm, out_hbm, buf0, buf1,
             si0, si1, so0, so1):
    wid = lax.axis_index("s") * 2 + lax.axis_index("c")
    bufs = (buf0, buf1)
    sin = (si0, si1)
    sout = (so0, so1)

    def _in(j, p):
        stripe = j // NCH
        col = (j % NCH) * CHUNK
        o = pl.multiple_of(stripe * 8, 8)
        return pltpu.make_async_copy(
            hist_hbm.at[pl.ds(o, 8), pl.ds(col, CHUNK)], bufs[p], sin[p])

    def _out(j, p):
        stripe = j // NCH
        col = (j % NCH) * CHUNK
        o = pl.multiple_of(stripe * 8, 8)
        return pltpu.make_async_copy(
            bufs[p], out_hbm.at[pl.ds(o, 8), pl.ds(col, CHUNK)], sout[p])

    def _fix_row0(j, p):
        @pl.when(j // NCH == 0)
        def _():
            col = (j % NCH) * CHUNK
            pltpu.async_copy(w_hbm.at[pl.ds(col, CHUNK)],
                             bufs[p].at[0], sin[p]).wait()

    @pl.when(wid < FULL_JOBS)
    def _prime():
        _in(wid, 0).start()

    def pair(g, carry):
        for p in range(NBUF):
            k = NBUF * g + p
            j = wid + NW * k

            @pl.when(j < FULL_JOBS)
            def _(j=j, p=p, k=k):
                _in(j, p).wait()
                _fix_row0(j, p)
                _out(j, p).start()
                q = 1 - p
                jprev = j - NW

                @pl.when(jprev >= 0)
                def _():
                    _out(jprev, q).wait()

                jnext = j + NW

                @pl.when(jnext < FULL_JOBS)
                def _():
                    _in(jnext, q).start()
        return carry

    npairs = (FULL_JOBS // NW + NBUF - 1) // NBUF
    lax.fori_loop(0, npairs, pair, 0)

    npairs_total = ((FULL_JOBS + NW - 1) // NW + NBUF - 1) // NBUF
    for k in (NBUF * npairs_total - 2, NBUF * npairs_total - 1):
        j_k = wid + NW * k

        @pl.when((j_k < FULL_JOBS) & (j_k + NW >= FULL_JOBS))
        def _drain(j_k=j_k, k=k):
            _out(j_k, k % NBUF).wait()

    def tail_body(k, carry):
        j = wid + NW * k

        @pl.when(j < NCH)
        def _():
            col = j * CHUNK
            pltpu.async_copy(
                hist_hbm.at[pl.ds(96, 4), pl.ds(col, CHUNK)],
                buf0.at[pl.ds(0, 4)], si0).wait()
            pltpu.async_copy(
                buf0.at[pl.ds(0, 4)],
                out_hbm.at[pl.ds(96, 4), pl.ds(col, CHUNK)], si0).wait()
        return carry

    lax.fori_loop(0, (NCH + NW - 1) // NW, tail_body, 0)


def _tc_body(g_ref, s_ref, t_ref, ge_ref, fe_ref,
             nge_ref, nfe_ref, u_ref, nge_s, nfe_s, smax):
    i = pl.program_id(0)

    @pl.when(i < NB)
    def _phase1():
        nge = DECAY * ge_ref[...] + (1.0 - DECAY) * jnp.abs(g_ref[...])
        nge_ref[...] = nge
        nge_s[pl.ds(i * B, B)] = nge
        bmax = jnp.max(nge)
        prev = jnp.where(i == 0, 0.0, smax[0])
        smax[0] = jnp.maximum(prev, bmax)

        s = s_ref[...]
        t = t_ref[...]
        sum_s = jnp.sum(s, axis=0)
        sum_t = jnp.sum(t, axis=0)
        sum_st = jnp.sum(s * t, axis=0)
        sum_ss = jnp.sum(s * s, axis=0)
        sum_tt = jnp.sum(t * t, axis=0)
        cov = sum_st - sum_s * sum_t * (1.0 / D)
        var_s = sum_ss - sum_s * sum_s * (1.0 / D)
        var_t = sum_tt - sum_t * sum_t * (1.0 / D)
        corr = cov / ((jnp.sqrt(var_s) + 1e-6) * (jnp.sqrt(var_t) + 1e-6))
        nfe = DECAY * fe_ref[...] + (1.0 - DECAY) * jnp.abs(corr)
        nfe_ref[...] = nfe
        nfe_s[pl.ds(i * B, B)] = nfe

    @pl.when(i >= NB)
    def _phase2():
        j = i - NB
        m = smax[0]
        u_ref[...] = (ALPHA / (m + 1e-6)) * nge_s[pl.ds(j * B, B)] \
            + (BETA * nfe_s[pl.ds(j * B, B)] + GAMMA)


def kernel(gradients, source_activations, target_activations, weights,
           gradient_ema, flow_ema, weight_history):
    sc_copy = pl.kernel(
        _sc_hist,
        out_type=jax.ShapeDtypeStruct((HIST, N), jnp.float32),
        mesh=plsc.VectorSubcoreMesh(core_axis_name="c", subcore_axis_name="s"),
        scratch_types=[pltpu.VMEM((8, CHUNK), jnp.float32),
                       pltpu.VMEM((8, CHUNK), jnp.float32),
                       pltpu.SemaphoreType.DMA, pltpu.SemaphoreType.DMA,
                       pltpu.SemaphoreType.DMA, pltpu.SemaphoreType.DMA],
    )
    nhist = sc_copy(weight_history, weights)

    sT = source_activations.T
    tT = target_activations.T

    clamp = lambda i: (jnp.minimum(i, NB - 1),)
    clamp2 = lambda i: (0, jnp.minimum(i, NB - 1))
    nge, nfe, utility = pl.pallas_call(
        _tc_body,
        grid=(2 * NB,),
        in_specs=[
            pl.BlockSpec((B,), clamp),
            pl.BlockSpec((D, B), clamp2),
            pl.BlockSpec((D, B), clamp2),
            pl.BlockSpec((B,), clamp),
            pl.BlockSpec((B,), clamp),
        ],
        out_specs=[
            pl.BlockSpec((B,), clamp),
            pl.BlockSpec((B,), clamp),
            pl.BlockSpec((B,), lambda i: (jnp.maximum(i - NB, 0),)),
        ],
        out_shape=[
            jax.ShapeDtypeStruct((N,), jnp.float32),
            jax.ShapeDtypeStruct((N,), jnp.float32),
            jax.ShapeDtypeStruct((N,), jnp.float32),
        ],
        scratch_shapes=[
            pltpu.VMEM((N,), jnp.float32),
            pltpu.VMEM((N,), jnp.float32),
            pltpu.SMEM((1,), jnp.float32),
        ],
    )(gradients, sT, tT, gradient_ema, flow_ema)

    return (utility, nge, nfe, nhist)


# speedup vs baseline: 1.0644x; 1.0022x over previous
import jax
import jax.numpy as jnp
from jax import lax
from jax.experimental import pallas as pl
from jax.experimental.pallas import tpu as pltpu
from jax.experimental.pallas import tpu_sc as plsc

N = 640000
D = 16
HIST = 100
ALPHA = 0.4
BETA = 0.4
GAMMA = 0.2
DECAY = 0.99

B = 25600
NB = N // B

NW = 32
CHUNK = 6400
NCH = N // CHUNK
NFULL = 12
FULL_JOBS = NFULL * NCH
NBUF = 2


def _sc_hist(hist_hbm, w_hbm, out_hbm, buf0, buf1,
             si0, si1, so0, so1):
    wid = lax.axis_index("s") * 2 + lax.axis_index("c")
    bufs = (buf0, buf1)
    sin = (si0, si1)
    sout = (so0, so1)

    def _in(j, p):
        stripe = j // NCH
        col = (j % NCH) * CHUNK
        o = pl.multiple_of(stripe * 8, 8)
        return pltpu.make_async_copy(
            hist_hbm.at[pl.ds(o, 8), pl.ds(col, CHUNK)], bufs[p], sin[p])

    def _out(j, p):
        stripe = j // NCH
        col = (j % NCH) * CHUNK
        o = pl.multiple_of(stripe * 8, 8)
        return pltpu.make_async_copy(
            bufs[p], out_hbm.at[pl.ds(o, 8), pl.ds(col, CHUNK)], sout[p])

    def _fix_row0(j, p):
        @pl.when(j // NCH == 0)
        def _():
            col = (j % NCH) * CHUNK
            pltpu.async_copy(w_hbm.at[pl.ds(col, CHUNK)],
                             bufs[p].at[0], sin[p]).wait()

    @pl.when(wid < FULL_JOBS)
    def _prime():
        _in(wid, 0).start()

    def pair(g, carry):
        for p in range(NBUF):
            k = NBUF * g + p
            j = wid + NW * k

            @pl.when(j < FULL_JOBS)
            def _(j=j, p=p, k=k):
                _in(j, p).wait()
                _fix_row0(j, p)
                _out(j, p).start()
                q = 1 - p
                jprev = j - NW

                @pl.when(jprev >= 0)
                def _():
                    _out(jprev, q).wait()

                jnext = j + NW

                @pl.when(jnext < FULL_JOBS)
                def _():
                    _in(jnext, q).start()
        return carry

    npairs = (FULL_JOBS // NW + NBUF - 1) // NBUF
    lax.fori_loop(0, npairs, pair, 0)

    npairs_total = ((FULL_JOBS + NW - 1) // NW + NBUF - 1) // NBUF
    for k in (NBUF * npairs_total - 2, NBUF * npairs_total - 1):
        j_k = wid + NW * k

        @pl.when((j_k < FULL_JOBS) & (j_k + NW >= FULL_JOBS))
        def _drain(j_k=j_k, k=k):
            _out(j_k, k % NBUF).wait()

    def tail_body(k, carry):
        j = wid + NW * k

        @pl.when(j < NCH)
        def _():
            col = j * CHUNK
            pltpu.async_copy(
                hist_hbm.at[pl.ds(96, 4), pl.ds(col, CHUNK)],
                buf0.at[pl.ds(0, 4)], si0).wait()
            pltpu.async_copy(
                buf0.at[pl.ds(0, 4)],
                out_hbm.at[pl.ds(96, 4), pl.ds(col, CHUNK)], si0).wait()
        return carry

    lax.fori_loop(0, (NCH + NW - 1) // NW, tail_body, 0)


def _tc_body(g_ref, s_ref, t_ref, ge_ref, fe_ref,
             nge_ref, nfe_ref, u_ref, nge_s, nfe_s, smax):
    i = pl.program_id(0)

    @pl.when(i < NB)
    def _phase1():
        nge = DECAY * ge_ref[...] + (1.0 - DECAY) * jnp.abs(g_ref[...])
        nge_ref[...] = nge
        nge_s[pl.ds(i * B, B)] = nge
        bmax = jnp.max(nge)
        prev = jnp.where(i == 0, 0.0, smax[0])
        smax[0] = jnp.maximum(prev, bmax)

        s = s_ref[...]
        t = t_ref[...]
        sum_s = jnp.sum(s, axis=0)
        sum_t = jnp.sum(t, axis=0)
        sum_st = jnp.sum(s * t, axis=0)
        sum_ss = jnp.sum(s * s, axis=0)
        sum_tt = jnp.sum(t * t, axis=0)
        cov = sum_st - sum_s * sum_t * (1.0 / D)
        var_s = sum_ss - sum_s * sum_s * (1.0 / D)
        var_t = sum_tt - sum_t * sum_t * (1.0 / D)
        corr = cov / ((jnp.sqrt(var_s) + 1e-6) * (jnp.sqrt(var_t) + 1e-6))
        nfe = DECAY * fe_ref[...] + (1.0 - DECAY) * jnp.abs(corr)
        nfe_ref[...] = nfe
        nfe_s[pl.ds(i * B, B)] = nfe

    @pl.when(i >= NB)
    def _phase2():
        j = i - NB
        m = smax[0]
        u_ref[...] = (ALPHA / (m + 1e-6)) * nge_s[pl.ds(j * B, B)] \
            + (BETA * nfe_s[pl.ds(j * B, B)] + GAMMA)


def kernel(gradients, source_activations, target_activations, weights,
           gradient_ema, flow_ema, weight_history):
    sc_copy = pl.kernel(
        _sc_hist,
        out_type=jax.ShapeDtypeStruct((HIST, N), jnp.float32),
        mesh=plsc.VectorSubcoreMesh(core_axis_name="c", subcore_axis_name="s"),
        scratch_types=[pltpu.VMEM((8, CHUNK), jnp.float32),
                       pltpu.VMEM((8, CHUNK), jnp.float32),
                       pltpu.SemaphoreType.DMA, pltpu.SemaphoreType.DMA,
                       pltpu.SemaphoreType.DMA, pltpu.SemaphoreType.DMA],
    )
    nhist = sc_copy(weight_history, weights)

    sT = source_activations.T
    tT = target_activations.T

    clamp = lambda i: (jnp.minimum(i, NB - 1),)
    clamp2 = lambda i: (0, jnp.minimum(i, NB - 1))
    nge, nfe, utility = pl.pallas_call(
        _tc_body,
        grid=(2 * NB,),
        in_specs=[
            pl.BlockSpec((B,), clamp),
            pl.BlockSpec((D, B), clamp2),
            pl.BlockSpec((D, B), clamp2),
            pl.BlockSpec((B,), clamp),
            pl.BlockSpec((B,), clamp),
        ],
        out_specs=[
            pl.BlockSpec((B,), clamp),
            pl.BlockSpec((B,), clamp),
            pl.BlockSpec((B,), lambda i: (jnp.maximum(i - NB, 0),)),
        ],
        out_shape=[
            jax.ShapeDtypeStruct((N,), jnp.float32),
            jax.ShapeDtypeStruct((N,), jnp.float32),
            jax.ShapeDtypeStruct((N,), jnp.float32),
        ],
        scratch_shapes=[
            pltpu.VMEM((N,), jnp.float32),
            pltpu.VMEM((N,), jnp.float32),
            pltpu.SMEM((1,), jnp.float32),
        ],
    )(gradients, sT, tT, gradient_ema, flow_ema)

    return (utility, nge, nfe, nhist)
